# baseline (speedup 1.0000x reference)
import functools

import jax
import jax.numpy as jnp
from jax import lax
from jax.experimental import pallas as pl
from jax.experimental.pallas import tpu as pltpu
from jax.experimental.pallas import tpu_sc as plsc

BATCH = 4
SEQ = 8192
HID = 2048
LANES = 16
UNROLL = 8
CHUNKS = SEQ // LANES
HALF_HID = HID // 2
INT_MIN = -2147483648


def _sc_body(hid_hbm, pids_hbm, out_hbm, pids_v, row_v, sem0, semg):
    wid = lax.axis_index("s")
    if True:
        b = wid % BATCH
        half = wid // BATCH
        h0 = half * HALF_HID
        out_slice = out_hbm.at[b, pl.ds(0, 1), pl.ds(h0, HALF_HID)]

        cpg = pltpu.async_copy(
            hid_hbm.at[b, pl.ds(SEQ - 1, 1), pl.ds(h0, HALF_HID)],
            row_v,
            semg,
        )
        cp0 = pltpu.async_copy(pids_hbm.at[b], pids_v, sem0)
        cpg.wait()
        cpo = pltpu.async_copy(row_v, out_slice, semg)

        lane_iota = lax.iota(jnp.int32, LANES)

        def scan_body(i, carry):
            cur_max, cur_chunk = carry
            for u in range(UNROLL):
                c = i * UNROLL + u
                v = pids_v[pl.ds(c * LANES, LANES)]
                take = v > cur_max
                cur_max = jnp.where(take, v, cur_max)
                cur_chunk = jnp.where(take, c, cur_chunk)
            return (cur_max, cur_chunk)

        cp0.wait()
        cur_max, cur_chunk = lax.fori_loop(
            0, CHUNKS // UNROLL, scan_body,
            (jnp.full((LANES,), INT_MIN, jnp.int32),
             jnp.zeros((LANES,), jnp.int32)),
        )
        cur_idx = cur_chunk * LANES + lane_iota

        gdn = lax.GatherDimensionNumbers(
            offset_dims=(), collapsed_slice_dims=(0,), start_index_map=(0,)
        )
        for s in (8, 4, 2, 1):
            pidx = lane_iota ^ s
            ov = lax.gather(
                cur_max, pidx[:, None], gdn, (1,),
                mode=lax.GatherScatterMode.PROMISE_IN_BOUNDS,
            )
            oi = lax.gather(
                cur_idx, pidx[:, None], gdn, (1,),
                mode=lax.GatherScatterMode.PROMISE_IN_BOUNDS,
            )
            take = (ov > cur_max) | ((ov == cur_max) & (oi < cur_idx))
            cur_max = jnp.where(take, ov, cur_max)
            cur_idx = jnp.where(take, oi, cur_idx)
        idx = cur_idx[0]

        cpo.wait()

        @pl.when(idx != SEQ - 1)
        def _():
            pltpu.sync_copy(
                hid_hbm.at[b, pl.ds(idx, 1), pl.ds(h0, HALF_HID)], out_slice
            )


@jax.jit
def _sc_kernel(hidden_states, position_ids):
    return pl.kernel(
        _sc_body,
        mesh=plsc.VectorSubcoreMesh(core_axis_name="c", subcore_axis_name="s", num_cores=1, num_subcores=8),
        out_type=jax.ShapeDtypeStruct((BATCH, 1, HID), jnp.float32),
        scratch_types=[
            pltpu.VMEM((SEQ,), jnp.int32),
            pltpu.VMEM((1, HALF_HID), jnp.float32),
            pltpu.SemaphoreType.DMA,
            pltpu.SemaphoreType.DMA,
        ],
    )(hidden_states, position_ids)


def kernel(hidden_states, position_ids):
    return _sc_kernel(hidden_states, position_ids)


# speedup vs baseline: 1.0004x; 1.0004x over previous
import jax
import jax.numpy as jnp
from jax import lax
from jax.experimental import pallas as pl
from jax.experimental.pallas import tpu as pltpu
from jax.experimental.pallas import tpu_sc as plsc

BATCH = 4
SEQ = 8192
HID = 2048
LANES = 16
UNROLL = 8
CHUNKS = SEQ // LANES
HALF_HID = HID // 2
INT_MIN = -2147483648


def _sc_body(hid_hbm, pids_hbm, out_hbm, pids_v, row_v, sem0, semg):
    wid = lax.axis_index("s")
    b = wid % BATCH
    half = wid // BATCH
    h0 = half * HALF_HID
    out_slice = out_hbm.at[b, pl.ds(0, 1), pl.ds(h0, HALF_HID)]

    cpg = pltpu.async_copy(
        hid_hbm.at[b, pl.ds(SEQ - 1, 1), pl.ds(h0, HALF_HID)],
        row_v,
        semg,
    )
    cp0 = pltpu.async_copy(pids_hbm.at[b], pids_v, sem0)
    cpg.wait()
    cpo = pltpu.async_copy(row_v, out_slice, semg)

    lane_iota = lax.iota(jnp.int32, LANES)

    def scan_body(i, carry):
        cur_max, cur_chunk = carry
        for u in range(UNROLL):
            c = i * UNROLL + u
            v = pids_v[pl.ds(c * LANES, LANES)]
            take = v > cur_max
            cur_max = jnp.where(take, v, cur_max)
            cur_chunk = jnp.where(take, c, cur_chunk)
        return (cur_max, cur_chunk)

    cp0.wait()
    cur_max, cur_chunk = lax.fori_loop(
        0, CHUNKS // UNROLL, scan_body,
        (jnp.full((LANES,), INT_MIN, jnp.int32),
         jnp.zeros((LANES,), jnp.int32)),
    )
    cur_idx = cur_chunk * LANES + lane_iota

    gdn = lax.GatherDimensionNumbers(
        offset_dims=(), collapsed_slice_dims=(0,), start_index_map=(0,)
    )
    for s in (8, 4, 2, 1):
        pidx = lane_iota ^ s
        ov = lax.gather(
            cur_max, pidx[:, None], gdn, (1,),
            mode=lax.GatherScatterMode.PROMISE_IN_BOUNDS,
        )
        oi = lax.gather(
            cur_idx, pidx[:, None], gdn, (1,),
            mode=lax.GatherScatterMode.PROMISE_IN_BOUNDS,
        )
        take = (ov > cur_max) | ((ov == cur_max) & (oi < cur_idx))
        cur_max = jnp.where(take, ov, cur_max)
        cur_idx = jnp.where(take, oi, cur_idx)
    idx = cur_idx[0]

    cpo.wait()

    @pl.when(idx != SEQ - 1)
    def _():
        pltpu.sync_copy(
            hid_hbm.at[b, pl.ds(idx, 1), pl.ds(h0, HALF_HID)], out_slice
        )


@jax.jit
def _sc_kernel(hidden_states, position_ids):
    return pl.kernel(
        _sc_body,
        mesh=plsc.VectorSubcoreMesh(
            core_axis_name="c", subcore_axis_name="s",
            num_cores=1, num_subcores=2 * BATCH,
        ),
        out_type=jax.ShapeDtypeStruct((BATCH, 1, HID), jnp.float32),
        scratch_types=[
            pltpu.VMEM((SEQ,), jnp.int32),
            pltpu.VMEM((1, HALF_HID), jnp.float32),
            pltpu.SemaphoreType.DMA,
            pltpu.SemaphoreType.DMA,
        ],
    )(hidden_states, position_ids)


def kernel(hidden_states, position_ids):
    return _sc_kernel(hidden_states, position_ids)


# speedup vs baseline: 1.0030x; 1.0027x over previous
import jax
import jax.numpy as jnp
from jax import lax
from jax.experimental import pallas as pl
from jax.experimental.pallas import tpu as pltpu
from jax.experimental.pallas import tpu_sc as plsc

BATCH = 4
SEQ = 8192
HID = 2048
LANES = 16
UNROLL = 8
CHUNKS = SEQ // LANES
HALF_HID = HID // 2
INT_MIN = -2147483648


def _sc_body(hid_hbm, pids_hbm, out_hbm, pids_v, row_v, sem0, semg):
    wid = lax.axis_index("s")
    b = wid
    h0 = 0
    out_slice = out_hbm.at[b, pl.ds(0, 1), pl.ds(h0, HID)]

    cpg = pltpu.async_copy(
        hid_hbm.at[b, pl.ds(SEQ - 1, 1), pl.ds(h0, HID)],
        row_v,
        semg,
    )
    cp0 = pltpu.async_copy(pids_hbm.at[b], pids_v, sem0)
    cpg.wait()
    cpo = pltpu.async_copy(row_v, out_slice, semg)

    lane_iota = lax.iota(jnp.int32, LANES)

    def scan_body(i, carry):
        cur_max, cur_chunk = carry
        for u in range(UNROLL):
            c = i * UNROLL + u
            v = pids_v[pl.ds(c * LANES, LANES)]
            take = v > cur_max
            cur_max = jnp.where(take, v, cur_max)
            cur_chunk = jnp.where(take, c, cur_chunk)
        return (cur_max, cur_chunk)

    cp0.wait()
    cur_max, cur_chunk = lax.fori_loop(
        0, CHUNKS // UNROLL, scan_body,
        (jnp.full((LANES,), INT_MIN, jnp.int32),
         jnp.zeros((LANES,), jnp.int32)),
    )
    cur_idx = cur_chunk * LANES + lane_iota

    gdn = lax.GatherDimensionNumbers(
        offset_dims=(), collapsed_slice_dims=(0,), start_index_map=(0,)
    )
    for s in (8, 4, 2, 1):
        pidx = lane_iota ^ s
        ov = lax.gather(
            cur_max, pidx[:, None], gdn, (1,),
            mode=lax.GatherScatterMode.PROMISE_IN_BOUNDS,
        )
        oi = lax.gather(
            cur_idx, pidx[:, None], gdn, (1,),
            mode=lax.GatherScatterMode.PROMISE_IN_BOUNDS,
        )
        take = (ov > cur_max) | ((ov == cur_max) & (oi < cur_idx))
        cur_max = jnp.where(take, ov, cur_max)
        cur_idx = jnp.where(take, oi, cur_idx)
    idx = cur_idx[0]

    cpo.wait()

    @pl.when(idx != SEQ - 1)
    def _():
        pltpu.sync_copy(
            hid_hbm.at[b, pl.ds(idx, 1), pl.ds(h0, HID)], out_slice
        )


@jax.jit
def _sc_kernel(hidden_states, position_ids):
    return pl.kernel(
        _sc_body,
        mesh=plsc.VectorSubcoreMesh(
            core_axis_name="c", subcore_axis_name="s",
            num_cores=1, num_subcores=BATCH,
        ),
        out_type=jax.ShapeDtypeStruct((BATCH, 1, HID), jnp.float32),
        scratch_types=[
            pltpu.VMEM((SEQ,), jnp.int32),
            pltpu.VMEM((1, HID), jnp.float32),
            pltpu.SemaphoreType.DMA,
            pltpu.SemaphoreType.DMA,
        ],
    )(hidden_states, position_ids)


def kernel(hidden_states, position_ids):
    return _sc_kernel(hidden_states, position_ids)


# speedup vs baseline: 1.0107x; 1.0077x over previous
import jax
import jax.numpy as jnp
from jax import lax
from jax.experimental import pallas as pl
from jax.experimental.pallas import tpu as pltpu
from jax.experimental.pallas import tpu_sc as plsc

BATCH = 4
SEQ = 8192
HID = 2048
LANES = 16
UNROLL = 4
CHUNKS = SEQ // LANES
HALF_HID = HID // 2
INT_MIN = -2147483648


def _sc_body(hid_hbm, pids_hbm, out_hbm, pids_v, row_v, sem0, semg):
    wid = lax.axis_index("s")
    b = wid
    h0 = 0
    out_slice = out_hbm.at[b, pl.ds(0, 1), pl.ds(h0, HID)]

    cpg = pltpu.async_copy(
        hid_hbm.at[b, pl.ds(SEQ - 1, 1), pl.ds(h0, HID)],
        row_v,
        semg,
    )
    cp0 = pltpu.async_copy(pids_hbm.at[b], pids_v, sem0)
    cpg.wait()
    cpo = pltpu.async_copy(row_v, out_slice, semg)

    lane_iota = lax.iota(jnp.int32, LANES)

    def scan_body(i, carry):
        cur_max, cur_chunk = carry
        for u in range(UNROLL):
            c = i * UNROLL + u
            v = pids_v[pl.ds(c * LANES, LANES)]
            take = v > cur_max
            cur_max = jnp.where(take, v, cur_max)
            cur_chunk = jnp.where(take, c, cur_chunk)
        return (cur_max, cur_chunk)

    cp0.wait()
    cur_max, cur_chunk = lax.fori_loop(
        0, CHUNKS // UNROLL, scan_body,
        (jnp.full((LANES,), INT_MIN, jnp.int32),
         jnp.zeros((LANES,), jnp.int32)),
    )
    cur_idx = cur_chunk * LANES + lane_iota

    gdn = lax.GatherDimensionNumbers(
        offset_dims=(), collapsed_slice_dims=(0,), start_index_map=(0,)
    )
    for s in (8, 4, 2, 1):
        pidx = lane_iota ^ s
        ov = lax.gather(
            cur_max, pidx[:, None], gdn, (1,),
            mode=lax.GatherScatterMode.PROMISE_IN_BOUNDS,
        )
        oi = lax.gather(
            cur_idx, pidx[:, None], gdn, (1,),
            mode=lax.GatherScatterMode.PROMISE_IN_BOUNDS,
        )
        take = (ov > cur_max) | ((ov == cur_max) & (oi < cur_idx))
        cur_max = jnp.where(take, ov, cur_max)
        cur_idx = jnp.where(take, oi, cur_idx)
    idx = cur_idx[0]

    cpo.wait()

    @pl.when(idx != SEQ - 1)
    def _():
        pltpu.sync_copy(
            hid_hbm.at[b, pl.ds(idx, 1), pl.ds(h0, HID)], out_slice
        )


@jax.jit
def _sc_kernel(hidden_states, position_ids):
    return pl.kernel(
        _sc_body,
        mesh=plsc.VectorSubcoreMesh(
            core_axis_name="c", subcore_axis_name="s",
            num_cores=1, num_subcores=BATCH,
        ),
        out_type=jax.ShapeDtypeStruct((BATCH, 1, HID), jnp.float32),
        scratch_types=[
            pltpu.VMEM((SEQ,), jnp.int32),
            pltpu.VMEM((1, HID), jnp.float32),
            pltpu.SemaphoreType.DMA,
            pltpu.SemaphoreType.DMA,
        ],
    )(hidden_states, position_ids)


def kernel(hidden_states, position_ids):
    return _sc_kernel(hidden_states, position_ids)
